# Initial kernel scaffold; baseline (speedup 1.0000x reference)
import jax
import jax.numpy as jnp
from jax.experimental import pallas as pl


def kernel(word_table, pos1_table, pos2_table, word, pos1, pos2, entity1, entity2):
    raise NotImplementedError("write your pallas kernel here")


# speedup vs baseline: 4.4196x; 4.4196x over previous
import functools

import jax
import jax.numpy as jnp
from jax import lax
from jax.experimental import pallas as pl
from jax.experimental.pallas import tpu as pltpu
from jax.experimental.pallas import tpu_sc as plsc

WORD_DIM = 32
POS_DIM = 16
EMB_DIM = WORD_DIM + 2 * POS_DIM

_NC = 2
_NS = 16
_NW = _NC * _NS

_CHUNK = 512


def _sc_body(n_per_w, e_per_w, n_chunks,
             word_table, pos1_table, pos2_table,
             word_idx, pos1_idx, pos2_idx, ent1, ent2,
             emb_out, word_out, ent1_out, ent2_out,
             widx_v, p1idx_v, p2idx_v, wrows_v, p1rows_v, p2rows_v,
             eidx_v, erows_v, sem0, sem1, sem2):
  wid = lax.axis_index("s") * _NC + lax.axis_index("c")
  base_w = wid * n_per_w

  def chunk_body(i, carry):
    base = base_w + i * _CHUNK
    pltpu.sync_copy(word_idx.at[pl.ds(base, _CHUNK)], widx_v)
    pltpu.sync_copy(pos1_idx.at[pl.ds(base, _CHUNK)], p1idx_v)
    pltpu.sync_copy(pos2_idx.at[pl.ds(base, _CHUNK)], p2idx_v)
    cw = pltpu.async_copy(word_table.at[widx_v], wrows_v, sem0)
    c1 = pltpu.async_copy(pos1_table.at[p1idx_v], p1rows_v, sem1)
    c2 = pltpu.async_copy(pos2_table.at[p2idx_v], p2rows_v, sem2)
    cw.wait()
    c1.wait()
    c2.wait()
    pltpu.sync_copy(wrows_v, word_out.at[pl.ds(base, _CHUNK)])
    pltpu.sync_copy(wrows_v, emb_out.at[pl.ds(base, _CHUNK), pl.ds(0, WORD_DIM)])
    pltpu.sync_copy(p1rows_v, emb_out.at[pl.ds(base, _CHUNK), pl.ds(WORD_DIM, POS_DIM)])
    pltpu.sync_copy(p2rows_v, emb_out.at[pl.ds(base, _CHUNK), pl.ds(WORD_DIM + POS_DIM, POS_DIM)])
    return carry

  lax.fori_loop(0, n_chunks, chunk_body, 0)

  ebase = wid * e_per_w
  pltpu.sync_copy(ent1.at[pl.ds(ebase, e_per_w)], eidx_v)
  pltpu.async_copy(word_table.at[eidx_v], erows_v, sem0).wait()
  pltpu.sync_copy(erows_v, ent1_out.at[pl.ds(ebase, e_per_w)])
  pltpu.sync_copy(ent2.at[pl.ds(ebase, e_per_w)], eidx_v)
  pltpu.async_copy(word_table.at[eidx_v], erows_v, sem0).wait()
  pltpu.sync_copy(erows_v, ent2_out.at[pl.ds(ebase, e_per_w)])


def kernel(word_table, pos1_table, pos2_table, word, pos1, pos2, entity1, entity2):
  B, L = word.shape
  N = B * L
  assert N % (_NW * _CHUNK) == 0
  n_per_w = N // _NW
  n_chunks = n_per_w // _CHUNK
  E = entity1.shape[0]
  e_per_w = E // _NW

  word_f = word.reshape(N).astype(jnp.int32)
  pos1_f = pos1.reshape(N).astype(jnp.int32)
  pos2_f = pos2.reshape(N).astype(jnp.int32)
  ent1 = entity1.astype(jnp.int32)
  ent2 = entity2.astype(jnp.int32)

  mesh = plsc.VectorSubcoreMesh(core_axis_name="c", subcore_axis_name="s")
  body = functools.partial(_sc_body, n_per_w, e_per_w, n_chunks)
  emb, word_e, ent1_e, ent2_e = pl.kernel(
      body,
      out_type=(
          jax.ShapeDtypeStruct((N, EMB_DIM), jnp.float32),
          jax.ShapeDtypeStruct((N, WORD_DIM), jnp.float32),
          jax.ShapeDtypeStruct((E, WORD_DIM), jnp.float32),
          jax.ShapeDtypeStruct((E, WORD_DIM), jnp.float32),
      ),
      mesh=mesh,
      compiler_params=pltpu.CompilerParams(use_tc_tiling_on_sc=False),
      scratch_types=[
          pltpu.VMEM((_CHUNK,), jnp.int32),
          pltpu.VMEM((_CHUNK,), jnp.int32),
          pltpu.VMEM((_CHUNK,), jnp.int32),
          pltpu.VMEM((_CHUNK, WORD_DIM), jnp.float32),
          pltpu.VMEM((_CHUNK, POS_DIM), jnp.float32),
          pltpu.VMEM((_CHUNK, POS_DIM), jnp.float32),
          pltpu.VMEM((e_per_w,), jnp.int32),
          pltpu.VMEM((e_per_w, WORD_DIM), jnp.float32),
          pltpu.SemaphoreType.DMA,
          pltpu.SemaphoreType.DMA,
          pltpu.SemaphoreType.DMA,
      ],
  )(word_table, pos1_table, pos2_table, word_f, pos1_f, pos2_f, ent1, ent2)

  embedding = emb.reshape(B, L, EMB_DIM)
  word_out = word_e.reshape(B, L, WORD_DIM)
  return (embedding, word_out, ent1_e, ent2_e)


# speedup vs baseline: 4.4647x; 1.0102x over previous
import functools

import jax
import jax.numpy as jnp
from jax import lax
from jax.experimental import pallas as pl
from jax.experimental.pallas import tpu as pltpu
from jax.experimental.pallas import tpu_sc as plsc

WORD_DIM = 32
POS_DIM = 16
EMB_DIM = WORD_DIM + 2 * POS_DIM

_NC = 2
_NS = 16
_NW = _NC * _NS

_CHUNK = 512


def _sc_body(n_per_w, e_per_w, n_pairs,
             word_table, pos1_table, pos2_table,
             word_idx, pos1_idx, pos2_idx, ent1, ent2,
             emb_out, word_out, ent1_out, ent2_out,
             widx0, p1idx0, p2idx0, wrows0, p1rows0, p2rows0,
             widx1, p1idx1, p2idx1, wrows1, p1rows1, p2rows1,
             eidx_v, erows_v,
             isem0, isem1, gsem0, gsem1, wsem0, wsem1):
  wid = lax.axis_index("s") * _NC + lax.axis_index("c")
  base_w = wid * n_per_w
  C = _CHUNK

  slots = (
      (widx0, p1idx0, p2idx0, wrows0, p1rows0, p2rows0, isem0, gsem0, wsem0),
      (widx1, p1idx1, p2idx1, wrows1, p1rows1, p2rows1, isem1, gsem1, wsem1),
  )

  def issue_idx(base, s):
    widx, p1idx, p2idx, _, _, _, isem, _, _ = slots[s]
    pltpu.async_copy(word_idx.at[pl.ds(base, C)], widx, isem)
    pltpu.async_copy(pos1_idx.at[pl.ds(base, C)], p1idx, isem)
    pltpu.async_copy(pos2_idx.at[pl.ds(base, C)], p2idx, isem)

  def drain_idx(s):
    widx, p1idx, p2idx, _, _, _, isem, _, _ = slots[s]
    pltpu.make_async_copy(word_idx.at[pl.ds(base_w, C)], widx, isem).wait()
    pltpu.make_async_copy(pos1_idx.at[pl.ds(base_w, C)], p1idx, isem).wait()
    pltpu.make_async_copy(pos2_idx.at[pl.ds(base_w, C)], p2idx, isem).wait()

  def issue_gathers(s):
    widx, p1idx, p2idx, wrows, p1rows, p2rows, _, gsem, _ = slots[s]
    pltpu.async_copy(word_table.at[widx], wrows, gsem)
    pltpu.async_copy(pos1_table.at[p1idx], p1rows, gsem)
    pltpu.async_copy(pos2_table.at[p2idx], p2rows, gsem)

  def drain_gathers(s):
    widx, p1idx, p2idx, wrows, p1rows, p2rows, _, gsem, _ = slots[s]
    pltpu.make_async_copy(word_table.at[widx], wrows, gsem).wait()
    pltpu.make_async_copy(pos1_table.at[p1idx], p1rows, gsem).wait()
    pltpu.make_async_copy(pos2_table.at[p2idx], p2rows, gsem).wait()

  def issue_writes(base, s):
    _, _, _, wrows, p1rows, p2rows, _, _, wsem = slots[s]
    pltpu.async_copy(wrows, word_out.at[pl.ds(base, C)], wsem)
    pltpu.async_copy(wrows, emb_out.at[pl.ds(base, C), pl.ds(0, WORD_DIM)], wsem)
    pltpu.async_copy(
        p1rows, emb_out.at[pl.ds(base, C), pl.ds(WORD_DIM, POS_DIM)], wsem)
    pltpu.async_copy(
        p2rows, emb_out.at[pl.ds(base, C), pl.ds(WORD_DIM + POS_DIM, POS_DIM)],
        wsem)

  def drain_writes(s):
    _, _, _, wrows, p1rows, p2rows, _, _, wsem = slots[s]
    pltpu.make_async_copy(wrows, word_out.at[pl.ds(base_w, C)], wsem).wait()
    pltpu.make_async_copy(
        wrows, emb_out.at[pl.ds(base_w, C), pl.ds(0, WORD_DIM)], wsem).wait()
    pltpu.make_async_copy(
        p1rows, emb_out.at[pl.ds(base_w, C), pl.ds(WORD_DIM, POS_DIM)],
        wsem).wait()
    pltpu.make_async_copy(
        p2rows, emb_out.at[pl.ds(base_w, C), pl.ds(WORD_DIM + POS_DIM, POS_DIM)],
        wsem).wait()

  issue_idx(base_w, 0)
  issue_idx(base_w + C, 1)

  def pair_body(j, carry):
    c0 = base_w + (2 * j) * C
    c1 = c0 + C

    @pl.when(j > 0)
    def _():
      drain_writes(0)
    drain_idx(0)
    issue_gathers(0)

    @pl.when(j > 0)
    def _():
      drain_writes(1)
    drain_idx(1)
    issue_gathers(1)

    drain_gathers(0)
    issue_writes(c0, 0)

    @pl.when(j < n_pairs - 1)
    def _():
      issue_idx(c0 + 2 * C, 0)

    drain_gathers(1)
    issue_writes(c1, 1)

    @pl.when(j < n_pairs - 1)
    def _():
      issue_idx(c1 + 2 * C, 1)
    return carry

  lax.fori_loop(0, n_pairs, pair_body, 0)
  drain_writes(0)
  drain_writes(1)

  ebase = wid * e_per_w
  pltpu.sync_copy(ent1.at[pl.ds(ebase, e_per_w)], eidx_v)
  pltpu.async_copy(word_table.at[eidx_v], erows_v, gsem0).wait()
  pltpu.sync_copy(erows_v, ent1_out.at[pl.ds(ebase, e_per_w)])
  pltpu.sync_copy(ent2.at[pl.ds(ebase, e_per_w)], eidx_v)
  pltpu.async_copy(word_table.at[eidx_v], erows_v, gsem0).wait()
  pltpu.sync_copy(erows_v, ent2_out.at[pl.ds(ebase, e_per_w)])


def kernel(word_table, pos1_table, pos2_table, word, pos1, pos2, entity1, entity2):
  B, L = word.shape
  N = B * L
  assert N % (_NW * 2 * _CHUNK) == 0
  n_per_w = N // _NW
  n_pairs = n_per_w // (2 * _CHUNK)
  E = entity1.shape[0]
  e_per_w = E // _NW

  word_f = word.reshape(N).astype(jnp.int32)
  pos1_f = pos1.reshape(N).astype(jnp.int32)
  pos2_f = pos2.reshape(N).astype(jnp.int32)
  ent1 = entity1.astype(jnp.int32)
  ent2 = entity2.astype(jnp.int32)

  mesh = plsc.VectorSubcoreMesh(core_axis_name="c", subcore_axis_name="s")
  body = functools.partial(_sc_body, n_per_w, e_per_w, n_pairs)
  slot_scratch = [
      pltpu.VMEM((_CHUNK,), jnp.int32),
      pltpu.VMEM((_CHUNK,), jnp.int32),
      pltpu.VMEM((_CHUNK,), jnp.int32),
      pltpu.VMEM((_CHUNK, WORD_DIM), jnp.float32),
      pltpu.VMEM((_CHUNK, POS_DIM), jnp.float32),
      pltpu.VMEM((_CHUNK, POS_DIM), jnp.float32),
  ]
  emb, word_e, ent1_e, ent2_e = pl.kernel(
      body,
      out_type=(
          jax.ShapeDtypeStruct((N, EMB_DIM), jnp.float32),
          jax.ShapeDtypeStruct((N, WORD_DIM), jnp.float32),
          jax.ShapeDtypeStruct((E, WORD_DIM), jnp.float32),
          jax.ShapeDtypeStruct((E, WORD_DIM), jnp.float32),
      ),
      mesh=mesh,
      compiler_params=pltpu.CompilerParams(use_tc_tiling_on_sc=False),
      scratch_types=slot_scratch + slot_scratch + [
          pltpu.VMEM((e_per_w,), jnp.int32),
          pltpu.VMEM((e_per_w, WORD_DIM), jnp.float32),
          pltpu.SemaphoreType.DMA,
          pltpu.SemaphoreType.DMA,
          pltpu.SemaphoreType.DMA,
          pltpu.SemaphoreType.DMA,
          pltpu.SemaphoreType.DMA,
          pltpu.SemaphoreType.DMA,
      ],
  )(word_table, pos1_table, pos2_table, word_f, pos1_f, pos2_f, ent1, ent2)

  embedding = emb.reshape(B, L, EMB_DIM)
  word_out = word_e.reshape(B, L, WORD_DIM)
  return (embedding, word_out, ent1_e, ent2_e)


# speedup vs baseline: 5.2959x; 1.1862x over previous
import functools

import jax
import jax.numpy as jnp
from jax import lax
from jax.experimental import pallas as pl
from jax.experimental.pallas import tpu as pltpu
from jax.experimental.pallas import tpu_sc as plsc

WORD_DIM = 32
POS_DIM = 16
EMB_DIM = WORD_DIM + 2 * POS_DIM
POS_VOCAB = 400

_NC = 2
_NS = 16
_NW = _NC * _NS

_CB = 512
_BT = _CB // 128


def _sc_body(B, L, n_units, n_pairs, e_per_w,
             word_table, p1t, p2t, wT, p1T, p2T, ent1, ent2,
             out5, w5, e1_5, e2_5,
             p1v, p2v,
             widx0, p1idx0, p2idx0, wrows0, slab0,
             widx1, p1idx1, p2idx1, wrows1, slab1,
             eidx, erows, eslab,
             isem0, isem1, gsem0, gsem1, wsem0, wsem1):
  wid = lax.axis_index("s") * _NC + lax.axis_index("c")
  nbt = B // 128
  cpl = B // _CB

  slots = (
      (widx0, p1idx0, p2idx0, wrows0, slab0, isem0, gsem0, wsem0),
      (widx1, p1idx1, p2idx1, wrows1, slab1, isem1, gsem1, wsem1),
  )

  pltpu.sync_copy(p1t, p1v)
  pltpu.sync_copy(p2t, p2v)

  iota16 = jax.lax.broadcasted_iota(jnp.int32, (16,), 0)

  def unit_lc(u):
    g = wid * n_units + u
    return g // cpl, g % cpl

  def issue_idx(u, s):
    widx, p1idx, p2idx, _, _, isem, _, _ = slots[s]
    l, c = unit_lc(u)
    b0 = c * _CB
    pltpu.async_copy(wT.at[l, pl.ds(b0, _CB)], widx, isem)
    pltpu.async_copy(p1T.at[l, pl.ds(b0, _CB)], p1idx, isem)
    pltpu.async_copy(p2T.at[l, pl.ds(b0, _CB)], p2idx, isem)

  def drain_idx(s):
    widx, p1idx, p2idx, _, _, isem, _, _ = slots[s]
    pltpu.make_async_copy(wT.at[0, pl.ds(0, _CB)], widx, isem).wait()
    pltpu.make_async_copy(p1T.at[0, pl.ds(0, _CB)], p1idx, isem).wait()
    pltpu.make_async_copy(p2T.at[0, pl.ds(0, _CB)], p2idx, isem).wait()

  def issue_gather(s):
    widx, _, _, wrows, _, _, gsem, _ = slots[s]
    pltpu.async_copy(word_table.at[widx], wrows, gsem)

  def drain_gather(s):
    widx, _, _, wrows, _, _, gsem, _ = slots[s]
    pltpu.make_async_copy(word_table.at[widx], wrows, gsem).wait()

  def issue_writes(u, s):
    _, _, _, _, slab, _, _, wsem = slots[s]
    l, c = unit_lc(u)
    bt0 = c * _BT
    pltpu.async_copy(slab, out5.at[l, :, pl.ds(bt0, _BT)], wsem)
    pltpu.async_copy(slab.at[pl.ds(0, WORD_DIM // 8)],
                     w5.at[l, :, pl.ds(bt0, _BT)], wsem)

  def drain_writes(s):
    _, _, _, _, slab, _, _, wsem = slots[s]
    pltpu.make_async_copy(slab, out5.at[0, :, pl.ds(0, _BT)], wsem).wait()
    pltpu.make_async_copy(slab.at[pl.ds(0, WORD_DIM // 8)],
                          w5.at[0, :, pl.ds(0, _BT)], wsem).wait()

  def tec_unit(s):
    _, p1idx, p2idx, wrows, slab, _, _, _ = slots[s]

    def g16_body(g16, carry):
      b0 = g16 * 16
      btp = g16 // 8
      bi0 = (g16 % 8) * 16
      rowi = iota16 + b0
      p1vec = p1idx[pl.ds(b0, 16)]
      p2vec = p2idx[pl.ds(b0, 16)]
      for d in range(WORD_DIM):
        vals = plsc.load_gather(
            wrows, [rowi, jnp.full((16,), d, jnp.int32)])
        slab[d // 8, btp, d % 8, pl.ds(bi0, 16)] = vals
      for d in range(POS_DIM):
        vals = plsc.load_gather(
            p1v, [jnp.full((16,), d, jnp.int32), p1vec])
        slab[4 + d // 8, btp, d % 8, pl.ds(bi0, 16)] = vals
      for d in range(POS_DIM):
        vals = plsc.load_gather(
            p2v, [jnp.full((16,), d, jnp.int32), p2vec])
        slab[6 + d // 8, btp, d % 8, pl.ds(bi0, 16)] = vals
      return carry

    lax.fori_loop(0, _CB // 16, g16_body, 0)

  issue_idx(0, 0)
  issue_idx(1, 1)
  drain_idx(0)
  issue_gather(0)
  drain_idx(1)
  issue_gather(1)

  def pair_body(j, carry):
    u0 = 2 * j
    u1 = u0 + 1

    @pl.when(j > 0)
    def _():
      drain_writes(0)
    drain_gather(0)
    tec_unit(0)
    issue_writes(u0, 0)

    @pl.when(j < n_pairs - 1)
    def _():
      issue_idx(u0 + 2, 0)

    @pl.when(j > 0)
    def _():
      drain_writes(1)
    drain_gather(1)

    @pl.when(j < n_pairs - 1)
    def _():
      drain_idx(0)
      issue_gather(0)

    tec_unit(1)
    issue_writes(u1, 1)

    @pl.when(j < n_pairs - 1)
    def _():
      issue_idx(u1 + 2, 1)
      drain_idx(1)
      issue_gather(1)
    return carry

  lax.fori_loop(0, n_pairs, pair_body, 0)
  drain_writes(0)
  drain_writes(1)

  ebase = wid * e_per_w
  for ent, eout in ((ent1, e1_5), (ent2, e2_5)):
    pltpu.sync_copy(ent.at[pl.ds(ebase, e_per_w)], eidx)
    pltpu.async_copy(word_table.at[eidx], erows, gsem0).wait()
    for g16 in range(e_per_w // 16):
      rowi = iota16 + g16 * 16
      for d in range(WORD_DIM):
        vals = plsc.load_gather(
            erows, [rowi, jnp.full((16,), d, jnp.int32)])
        eslab[d // 8, d % 8, pl.ds(g16 * 16, 16)] = vals
    pltpu.sync_copy(eslab, eout.at[:, wid])


def kernel(word_table, pos1_table, pos2_table, word, pos1, pos2, entity1, entity2):
  B, L = word.shape
  E = entity1.shape[0]
  assert (L * B) % (_NW * 2 * _CB) == 0 and B % _CB == 0 and E % (_NW * 128) == 0
  n_units = (L * B) // (_NW * _CB)
  n_pairs = n_units // 2
  e_per_w = E // _NW

  wT = word.T.astype(jnp.int32)
  p1T = pos1.T.astype(jnp.int32)
  p2T = pos2.T.astype(jnp.int32)
  p1t = pos1_table.T
  p2t = pos2_table.T
  ent1 = entity1.astype(jnp.int32)
  ent2 = entity2.astype(jnp.int32)

  mesh = plsc.VectorSubcoreMesh(core_axis_name="c", subcore_axis_name="s")
  body = functools.partial(_sc_body, B, L, n_units, n_pairs, e_per_w)
  out5, w5, e1_5, e2_5 = pl.kernel(
      body,
      out_type=(
          jax.ShapeDtypeStruct((L, EMB_DIM // 8, B // 128, 8, 128), jnp.float32),
          jax.ShapeDtypeStruct((L, WORD_DIM // 8, B // 128, 8, 128), jnp.float32),
          jax.ShapeDtypeStruct((WORD_DIM // 8, E // 128, 8, 128), jnp.float32),
          jax.ShapeDtypeStruct((WORD_DIM // 8, E // 128, 8, 128), jnp.float32),
      ),
      mesh=mesh,
      compiler_params=pltpu.CompilerParams(
          use_tc_tiling_on_sc=False, needs_layout_passes=False),
      scratch_types=[
          pltpu.VMEM((POS_DIM, POS_VOCAB), jnp.float32),
          pltpu.VMEM((POS_DIM, POS_VOCAB), jnp.float32),
          pltpu.VMEM((_CB,), jnp.int32),
          pltpu.VMEM((_CB,), jnp.int32),
          pltpu.VMEM((_CB,), jnp.int32),
          pltpu.VMEM((_CB, WORD_DIM), jnp.float32),
          pltpu.VMEM((EMB_DIM // 8, _BT, 8, 128), jnp.float32),
          pltpu.VMEM((_CB,), jnp.int32),
          pltpu.VMEM((_CB,), jnp.int32),
          pltpu.VMEM((_CB,), jnp.int32),
          pltpu.VMEM((_CB, WORD_DIM), jnp.float32),
          pltpu.VMEM((EMB_DIM // 8, _BT, 8, 128), jnp.float32),
          pltpu.VMEM((E // _NW,), jnp.int32),
          pltpu.VMEM((E // _NW, WORD_DIM), jnp.float32),
          pltpu.VMEM((WORD_DIM // 8, 8, 128), jnp.float32),
          pltpu.SemaphoreType.DMA,
          pltpu.SemaphoreType.DMA,
          pltpu.SemaphoreType.DMA,
          pltpu.SemaphoreType.DMA,
          pltpu.SemaphoreType.DMA,
          pltpu.SemaphoreType.DMA,
      ],
  )(word_table, p1t, p2t, wT, p1T, p2T, ent1, ent2)

  embedding = out5.transpose(2, 4, 0, 1, 3).reshape(B, L, EMB_DIM)
  word_out = w5.transpose(2, 4, 0, 1, 3).reshape(B, L, WORD_DIM)
  ent1_e = e1_5.transpose(1, 3, 0, 2).reshape(E, WORD_DIM)
  ent2_e = e2_5.transpose(1, 3, 0, 2).reshape(E, WORD_DIM)
  return (embedding, word_out, ent1_e, ent2_e)


# speedup vs baseline: 7.5070x; 1.4175x over previous
import functools

import jax
import jax.numpy as jnp
from jax import lax
from jax.experimental import pallas as pl
from jax.experimental.pallas import tpu as pltpu
from jax.experimental.pallas import tpu_sc as plsc

WORD_DIM = 32
POS_DIM = 16
EMB_DIM = WORD_DIM + 2 * POS_DIM
POS_VOCAB = 400

_NC = 2
_NS = 16
_NW = _NC * _NS

_CB = 512
_BT = _CB // 128


def _sc_body(B, L, n_units, n_pairs, e_per_w,
             word_table, p1t, p2t, wT, p1T, p2T, ent1, ent2,
             out5, w5, e1_5, e2_5,
             p1v, p2v,
             widx0, p1idx0, p2idx0, wrows0, slab0,
             widx1, p1idx1, p2idx1, wrows1, slab1,
             eidx, erows, eslab,
             isem0, isem1, gsem0, gsem1, wsem0, wsem1):
  wid = lax.axis_index("s") * _NC + lax.axis_index("c")
  nbt = B // 128
  cpl = B // _CB

  slots = (
      (widx0, p1idx0, p2idx0, wrows0, slab0, isem0, gsem0, wsem0),
      (widx1, p1idx1, p2idx1, wrows1, slab1, isem1, gsem1, wsem1),
  )

  pltpu.sync_copy(p1t, p1v)
  pltpu.sync_copy(p2t, p2v)

  iota16 = jax.lax.broadcasted_iota(jnp.int32, (16,), 0)

  def unit_lc(u):
    g = wid * n_units + u
    return g // cpl, g % cpl

  def issue_idx(u, s):
    widx, p1idx, p2idx, _, _, isem, _, _ = slots[s]
    l, c = unit_lc(u)
    b0 = c * _CB
    pltpu.async_copy(wT.at[l, pl.ds(b0, _CB)], widx, isem)
    pltpu.async_copy(p1T.at[l, pl.ds(b0, _CB)], p1idx, isem)
    pltpu.async_copy(p2T.at[l, pl.ds(b0, _CB)], p2idx, isem)

  def drain_idx(s):
    widx, p1idx, p2idx, _, _, isem, _, _ = slots[s]
    pltpu.make_async_copy(wT.at[0, pl.ds(0, _CB)], widx, isem).wait()
    pltpu.make_async_copy(p1T.at[0, pl.ds(0, _CB)], p1idx, isem).wait()
    pltpu.make_async_copy(p2T.at[0, pl.ds(0, _CB)], p2idx, isem).wait()

  def issue_gather(s):
    widx, _, _, wrows, _, _, gsem, _ = slots[s]
    pltpu.async_copy(word_table.at[widx], wrows, gsem)

  def drain_gather(s):
    widx, _, _, wrows, _, _, gsem, _ = slots[s]
    pltpu.make_async_copy(word_table.at[widx], wrows, gsem).wait()

  def issue_writes(u, s):
    _, _, _, _, slab, _, _, wsem = slots[s]
    l, c = unit_lc(u)
    bt0 = c * _BT
    pltpu.async_copy(slab, out5.at[l, :, pl.ds(bt0, _BT)], wsem)
    pltpu.async_copy(slab.at[pl.ds(0, WORD_DIM // 8)],
                     w5.at[l, :, pl.ds(bt0, _BT)], wsem)

  def drain_writes(s):
    _, _, _, _, slab, _, _, wsem = slots[s]
    pltpu.make_async_copy(slab, out5.at[0, :, pl.ds(0, _BT)], wsem).wait()
    pltpu.make_async_copy(slab.at[pl.ds(0, WORD_DIM // 8)],
                          w5.at[0, :, pl.ds(0, _BT)], wsem).wait()

  def tec_unit(s):
    _, p1idx, p2idx, wrows, slab, _, _, _ = slots[s]

    def g16_body(g16, carry):
      b0 = g16 * 16
      btp = g16 // 8
      bi0 = (g16 % 8) * 16
      rowi = iota16 + b0
      p1vec = p1idx[pl.ds(b0, 16)]
      p2vec = p2idx[pl.ds(b0, 16)]
      for h in range(WORD_DIM // 16):
        vals = [plsc.load_gather(
            wrows, [rowi, jnp.full((16,), 16 * h + e, jnp.int32)])
            for e in range(16)]
        for e in range(16):
          d = 16 * h + e
          slab[d // 8, btp, d % 8, pl.ds(bi0, 16)] = vals[e]
      vals = [plsc.load_gather(
          p1v, [jnp.full((16,), d, jnp.int32), p1vec])
          for d in range(POS_DIM)]
      for d in range(POS_DIM):
        slab[4 + d // 8, btp, d % 8, pl.ds(bi0, 16)] = vals[d]
      vals = [plsc.load_gather(
          p2v, [jnp.full((16,), d, jnp.int32), p2vec])
          for d in range(POS_DIM)]
      for d in range(POS_DIM):
        slab[6 + d // 8, btp, d % 8, pl.ds(bi0, 16)] = vals[d]
      return carry

    lax.fori_loop(0, _CB // 16, g16_body, 0)

  issue_idx(0, 0)
  issue_idx(1, 1)
  drain_idx(0)
  issue_gather(0)
  drain_idx(1)
  issue_gather(1)

  def pair_body(j, carry):
    u0 = 2 * j
    u1 = u0 + 1

    @pl.when(j > 0)
    def _():
      drain_writes(0)
    drain_gather(0)
    tec_unit(0)
    issue_writes(u0, 0)

    @pl.when(j < n_pairs - 1)
    def _():
      issue_idx(u0 + 2, 0)

    @pl.when(j > 0)
    def _():
      drain_writes(1)
    drain_gather(1)

    @pl.when(j < n_pairs - 1)
    def _():
      drain_idx(0)
      issue_gather(0)

    tec_unit(1)
    issue_writes(u1, 1)

    @pl.when(j < n_pairs - 1)
    def _():
      issue_idx(u1 + 2, 1)
      drain_idx(1)
      issue_gather(1)
    return carry

  lax.fori_loop(0, n_pairs, pair_body, 0)
  drain_writes(0)
  drain_writes(1)

  ebase = wid * e_per_w
  for ent, eout in ((ent1, e1_5), (ent2, e2_5)):
    pltpu.sync_copy(ent.at[pl.ds(ebase, e_per_w)], eidx)
    pltpu.async_copy(word_table.at[eidx], erows, gsem0).wait()
    for g16 in range(e_per_w // 16):
      rowi = iota16 + g16 * 16
      for h in range(WORD_DIM // 16):
        vals = [plsc.load_gather(
            erows, [rowi, jnp.full((16,), 16 * h + e, jnp.int32)])
            for e in range(16)]
        for e in range(16):
          d = 16 * h + e
          eslab[d // 8, d % 8, pl.ds(g16 * 16, 16)] = vals[e]
    pltpu.sync_copy(eslab, eout.at[:, wid])


def kernel(word_table, pos1_table, pos2_table, word, pos1, pos2, entity1, entity2):
  B, L = word.shape
  E = entity1.shape[0]
  assert (L * B) % (_NW * 2 * _CB) == 0 and B % _CB == 0 and E % (_NW * 128) == 0
  n_units = (L * B) // (_NW * _CB)
  n_pairs = n_units // 2
  e_per_w = E // _NW

  wT = word.T.astype(jnp.int32)
  p1T = pos1.T.astype(jnp.int32)
  p2T = pos2.T.astype(jnp.int32)
  p1t = pos1_table.T
  p2t = pos2_table.T
  ent1 = entity1.astype(jnp.int32)
  ent2 = entity2.astype(jnp.int32)

  mesh = plsc.VectorSubcoreMesh(core_axis_name="c", subcore_axis_name="s")
  body = functools.partial(_sc_body, B, L, n_units, n_pairs, e_per_w)
  out5, w5, e1_5, e2_5 = pl.kernel(
      body,
      out_type=(
          jax.ShapeDtypeStruct((L, EMB_DIM // 8, B // 128, 8, 128), jnp.float32),
          jax.ShapeDtypeStruct((L, WORD_DIM // 8, B // 128, 8, 128), jnp.float32),
          jax.ShapeDtypeStruct((WORD_DIM // 8, E // 128, 8, 128), jnp.float32),
          jax.ShapeDtypeStruct((WORD_DIM // 8, E // 128, 8, 128), jnp.float32),
      ),
      mesh=mesh,
      compiler_params=pltpu.CompilerParams(
          use_tc_tiling_on_sc=False, needs_layout_passes=False),
      scratch_types=[
          pltpu.VMEM((POS_DIM, POS_VOCAB), jnp.float32),
          pltpu.VMEM((POS_DIM, POS_VOCAB), jnp.float32),
          pltpu.VMEM((_CB,), jnp.int32),
          pltpu.VMEM((_CB,), jnp.int32),
          pltpu.VMEM((_CB,), jnp.int32),
          pltpu.VMEM((_CB, WORD_DIM), jnp.float32),
          pltpu.VMEM((EMB_DIM // 8, _BT, 8, 128), jnp.float32),
          pltpu.VMEM((_CB,), jnp.int32),
          pltpu.VMEM((_CB,), jnp.int32),
          pltpu.VMEM((_CB,), jnp.int32),
          pltpu.VMEM((_CB, WORD_DIM), jnp.float32),
          pltpu.VMEM((EMB_DIM // 8, _BT, 8, 128), jnp.float32),
          pltpu.VMEM((E // _NW,), jnp.int32),
          pltpu.VMEM((E // _NW, WORD_DIM), jnp.float32),
          pltpu.VMEM((WORD_DIM // 8, 8, 128), jnp.float32),
          pltpu.SemaphoreType.DMA,
          pltpu.SemaphoreType.DMA,
          pltpu.SemaphoreType.DMA,
          pltpu.SemaphoreType.DMA,
          pltpu.SemaphoreType.DMA,
          pltpu.SemaphoreType.DMA,
      ],
  )(word_table, p1t, p2t, wT, p1T, p2T, ent1, ent2)

  embedding = out5.transpose(2, 4, 0, 1, 3).reshape(B, L, EMB_DIM)
  word_out = w5.transpose(2, 4, 0, 1, 3).reshape(B, L, WORD_DIM)
  ent1_e = e1_5.transpose(1, 3, 0, 2).reshape(E, WORD_DIM)
  ent2_e = e2_5.transpose(1, 3, 0, 2).reshape(E, WORD_DIM)
  return (embedding, word_out, ent1_e, ent2_e)
